# baseline (speedup 1.0000x reference)
import dataclasses
import functools

import jax
import jax.numpy as jnp
from jax import lax
from jax.experimental import pallas as pl
from jax.experimental.pallas import tpu as pltpu
from jax.experimental.pallas import tpu_sc as plsc

NC = 2
NS = 16
NW = NC * NS
L = 16
K = 64
CHUNK = 128


def _sc_mf_kernel(B):
    b_per_w = B // NW
    n_chunks = b_per_w // CHUNK
    mesh = plsc.VectorSubcoreMesh(core_axis_name="c", subcore_axis_name="s")
    cp = pltpu.CompilerParams()
    if "needs_layout_passes" in pltpu.CompilerParams.__dataclass_fields__:
        cp = dataclasses.replace(cp, needs_layout_passes=False)
    if "use_tc_tiling_on_sc" in pltpu.CompilerParams.__dataclass_fields__:
        cp = dataclasses.replace(cp, use_tc_tiling_on_sc=False)

    @functools.partial(
        pl.kernel,
        out_type=jax.ShapeDtypeStruct((NW, b_per_w), jnp.float32),
        mesh=mesh,
        compiler_params=cp,
        scratch_types=[
            pltpu.VMEM((n_chunks, CHUNK), jnp.int32),
            pltpu.VMEM((n_chunks, CHUNK), jnp.int32),
            pltpu.VMEM((n_chunks, CHUNK, K), jnp.float32),
            pltpu.VMEM((n_chunks, CHUNK, K), jnp.float32),
            pltpu.VMEM((b_per_w,), jnp.float32),
            pltpu.SemaphoreType.DMA,
        ],
    )
    def kern(u_idx_hbm, i_idx_hbm, uf_hbm, if_hbm, out_hbm,
             u_idx, i_idx, u_rows, v_rows, out_v, sem):
        wid = lax.axis_index("s") * NC + lax.axis_index("c")

        pltpu.sync_copy(u_idx_hbm.at[wid], u_idx)
        pltpu.sync_copy(i_idx_hbm.at[wid], i_idx)

        copies = []
        for c in range(n_chunks):
            copies.append(
                pltpu.async_copy(uf_hbm.at[u_idx.at[c]], u_rows.at[c], sem))
            copies.append(
                pltpu.async_copy(if_hbm.at[i_idx.at[c]], v_rows.at[c], sem))
        for cp in copies:
            cp.wait()

        lane = lax.iota(jnp.int32, L)
        for c in range(n_chunks):
            @pl.loop(0, CHUNK, step=L)
            def _(r0, c=c):
                acc = jnp.zeros((L,), jnp.float32)
                for j in range(L):
                    r = r0 + j
                    s = (u_rows[c, r, pl.ds(0, L)] * v_rows[c, r, pl.ds(0, L)]
                         + u_rows[c, r, pl.ds(L, L)] * v_rows[c, r, pl.ds(L, L)]
                         + u_rows[c, r, pl.ds(2 * L, L)] * v_rows[c, r, pl.ds(2 * L, L)]
                         + u_rows[c, r, pl.ds(3 * L, L)] * v_rows[c, r, pl.ds(3 * L, L)])
                    acc = jnp.where(lane == j, jnp.sum(s), acc)
                out_v[pl.ds(c * CHUNK + r0, L)] = acc

        pltpu.sync_copy(out_v, out_hbm.at[wid])

    return kern


def kernel(user, item, user_factors, item_factors):
    B = user.shape[0]
    b_per_w = B // NW
    n_chunks = b_per_w // CHUNK
    u_idx = user.astype(jnp.int32).reshape(NW, n_chunks, CHUNK)
    i_idx = item.astype(jnp.int32).reshape(NW, n_chunks, CHUNK)
    out = _sc_mf_kernel(B)(u_idx, i_idx, user_factors, item_factors)
    return out.reshape(B)
